# baseline (speedup 1.0000x reference)
import functools

import jax
import jax.numpy as jnp
from jax import lax
from jax.experimental import pallas as pl
from jax.experimental.pallas import tpu as pltpu
from jax.experimental.pallas import tpu_sc as plsc

N = 10000
E = 160000
H = 128
L = 3
NG = 20
NT = 5
ED = 4
B = 64

NGP = 24
PACK = 144
GPACK = 160
N_PAD = 10240
E_PAD = 163840
DUMMY = N

NC = 2
NS = 16
NW = NC * NS
CHUNK = 128

CH = 2
E_C = E_PAD // CH
G_ROWS = 2 * E_C
G_CH_W = G_ROWS // NW // CHUNK
SCHUNK = 64
S_NBUF = 2
S_CH_W = E_C // NW // SCHUNK
ROWS_SUB = N_PAD // NS

EBLK = 1024
NBLK = 1024

def _mesh():
    return plsc.VectorSubcoreMesh(core_axis_name="c", subcore_axis_name="s")


NBUF = 4


GBUF = 8


def _sc_gather(table, idx2):
    @functools.partial(
        pl.kernel,
        out_type=jax.ShapeDtypeStruct((G_ROWS, GPACK), jnp.bfloat16),
        mesh=_mesh(),
        compiler_params=pltpu.CompilerParams(use_tc_tiling_on_sc=False),
        scratch_types=[
            pltpu.VMEM((G_CH_W * CHUNK,), jnp.int32),
        ] + [pltpu.VMEM((CHUNK, GPACK), jnp.bfloat16)] * GBUF
          + [pltpu.SemaphoreType.DMA] * (2 * GBUF),
    )
    def k(table_hbm, idx_hbm, out_hbm, idx_all, *rest):
        bufs = rest[:GBUF]
        gs = rest[GBUF:2 * GBUF]
        ws = rest[2 * GBUF:]
        wid = lax.axis_index("s") * NC + lax.axis_index("c")
        pltpu.sync_copy(idx_hbm.at[wid], idx_all)
        base_row = wid * G_CH_W * CHUNK

        def gidx(i):
            return idx_all.at[pl.ds(i * CHUNK, CHUNK)]

        def orow(i):
            return out_hbm.at[pl.ds(base_row + i * CHUNK, CHUNK)]

        def start_g(i, b):
            pltpu.async_copy(table_hbm.at[gidx(i)], bufs[b], gs[b])

        def wait_g(i, b):
            pltpu.make_async_copy(table_hbm.at[gidx(i)], bufs[b],
                                  gs[b]).wait()

        def start_w(i, b):
            pltpu.async_copy(bufs[b], orow(i), ws[b])

        def wait_w(i, b):
            pltpu.make_async_copy(bufs[b], orow(i), ws[b]).wait()

        for b in range(GBUF):
            start_g(b, b)

        @pl.loop(0, G_CH_W // GBUF - 1)
        def _(j):
            for b in range(GBUF):
                wait_g(j * GBUF + b, b)
                start_w(j * GBUF + b, b)
            for b in range(GBUF):
                wait_w(j * GBUF + b, b)
                start_g((j + 1) * GBUF + b, b)

        last = G_CH_W - GBUF
        for b in range(GBUF):
            wait_g(last + b, b)
            start_w(last + b, b)
        for b in range(GBUF):
            wait_w(last + b, b)

    return k(table, idx2)


def _sc_scatter(msg, dst3, zeros_tab):
    @functools.partial(
        pl.kernel,
        out_type=jax.ShapeDtypeStruct((NC, N_PAD, PACK), jnp.float32),
        mesh=_mesh(),
        compiler_params=pltpu.CompilerParams(use_tc_tiling_on_sc=False),
        scratch_types=[
            pltpu.VMEM((S_CH_W, SCHUNK), jnp.int32),
        ] + [pltpu.VMEM((SCHUNK, PACK), jnp.float32)] * S_NBUF
          + [pltpu.VMEM_SHARED((N_PAD, PACK), jnp.float32)]
          + [pltpu.SemaphoreType.DMA] * S_NBUF,
    )
    def k(msg_hbm, dst_hbm, zeros_hbm, out_hbm, idx_all, b0, b1,
          acc_sh, s0, s1):
        bufs = (b0, b1)
        sems = (s0, s1)
        c = lax.axis_index("c")
        s = lax.axis_index("s")
        wid = s * NC + c
        pltpu.sync_copy(zeros_hbm.at[pl.ds(s * ROWS_SUB, ROWS_SUB)],
                        acc_sh.at[pl.ds(s * ROWS_SUB, ROWS_SUB)])
        pltpu.sync_copy(dst_hbm.at[wid], idx_all)
        plsc.subcore_barrier()
        base_e = wid * S_CH_W * SCHUNK

        def mrow(i):
            return msg_hbm.at[pl.ds(base_e + i * SCHUNK, SCHUNK)]

        for b in range(S_NBUF):
            pltpu.async_copy(mrow(b), bufs[b], sems[b])

        def step(i, b):
            pltpu.make_async_copy(mrow(i), bufs[b], sems[b]).wait()
            pltpu.sync_copy(bufs[b], acc_sh.at[idx_all.at[i]], add=True)

        @pl.loop(0, S_CH_W // S_NBUF - 1)
        def _(j):
            for b in range(S_NBUF):
                i = j * S_NBUF + b
                step(i, b)
                pltpu.async_copy(mrow(i + S_NBUF), bufs[b], sems[b])

        for b in range(S_NBUF):
            step(S_CH_W - S_NBUF + b, b)

        plsc.subcore_barrier()
        pltpu.sync_copy(acc_sh.at[pl.ds(s * ROWS_SUB, ROWS_SUB)],
                        out_hbm.at[c].at[pl.ds(s * ROWS_SUB, ROWS_SUB)])

    return k(msg, dst3, zeros_tab)


def _pack_gather_rows(h, x):
    bf = jnp.bfloat16
    x_hi = x.astype(bf)
    x_lo = (x - x_hi.astype(jnp.float32)).astype(bf)
    pad = jnp.zeros((h.shape[0], GPACK - H - 6), bf)
    return jnp.concatenate([h.astype(bf), x_hi, x_lo, pad], axis=1)


def _init_body(remap_ref, ntf_ref, pos_ref, w0_ref, w1_ref, w2_ref, b_ref,
               out_ref, outg_ref):
    c = ntf_ref[...] / 9.0
    remap = remap_ref[...]
    h = (jnp.dot(remap, w0_ref[...], preferred_element_type=jnp.float32)
         + jnp.dot(remap * c, w1_ref[...], preferred_element_type=jnp.float32)
         + jnp.dot(remap * (c * c), w2_ref[...],
                   preferred_element_type=jnp.float32)
         + b_ref[...])
    pos = pos_ref[...]
    pad = jnp.zeros((out_ref.shape[0], PACK - H - 3), jnp.float32)
    out_ref[...] = jnp.concatenate([h, pos, pad], axis=1)
    outg_ref[...] = _pack_gather_rows(h, pos)


def _tc_init(remap_pad, ntf_pad, pos_pad, w0, w1, w2, b_node):
    grid = N_PAD // NBLK
    full = lambda shape: pl.BlockSpec(shape, lambda i: (0, 0))
    return pl.pallas_call(
        _init_body,
        grid=(grid,),
        in_specs=[
            pl.BlockSpec((NBLK, 8), lambda i: (i, 0)),
            pl.BlockSpec((NBLK, 1), lambda i: (i, 0)),
            pl.BlockSpec((NBLK, 3), lambda i: (i, 0)),
            full((8, H)), full((8, H)), full((8, H)), full((1, H)),
        ],
        out_specs=[pl.BlockSpec((NBLK, PACK), lambda i: (i, 0)),
                   pl.BlockSpec((NBLK, GPACK), lambda i: (i, 0))],
        out_shape=[jax.ShapeDtypeStruct((N_PAD, PACK), jnp.float32),
                   jax.ShapeDtypeStruct((N_PAD, GPACK), jnp.bfloat16)],
    )(remap_pad, ntf_pad, pos_pad, w0, w1, w2, b_node)


def _edge_body(gs_ref, gd_ref, et_ref, mu_ref, etab_ref,
               w1s_ref, w1d_ref, w1r_ref, w1w_ref, b1_ref,
               w2_ref, b2_ref, wx1_ref, bx1_ref, wx2_ref, bx2_ref,
               out_ref):
    gs = gs_ref[...]
    gd = gd_ref[...]
    f32 = jnp.float32
    hs = gs[:, :H].astype(f32)
    hd = gd[:, :H].astype(f32)
    xs = gs[:, H:H + 3].astype(f32) + gs[:, H + 3:H + 6].astype(f32)
    xd = gd[:, H:H + 3].astype(f32) + gd[:, H + 3:H + 6].astype(f32)
    dvec = xd - xs
    d = jnp.sqrt(jnp.sum(dvec * dvec, axis=1, keepdims=True) + 1e-8)
    sigma = 10.0 / NG
    rbf = jnp.exp(-((d - mu_ref[...]) ** 2) / (2.0 * sigma * sigma))

    et = et_ref[...]
    w = jnp.zeros((et.shape[0], H), jnp.float32)
    for kk in range(ED):
        w = w + jnp.where(et == float(kk), 1.0, 0.0) * etab_ref[kk:kk + 1, :]

    pre = (jnp.dot(hs, w1s_ref[...], preferred_element_type=jnp.float32)
           + jnp.dot(hd, w1d_ref[...], preferred_element_type=jnp.float32)
           + jnp.dot(rbf, w1r_ref[...], preferred_element_type=jnp.float32)
           + jnp.dot(w, w1w_ref[...], preferred_element_type=jnp.float32)
           + b1_ref[...])
    m = (jnp.dot(jax.nn.relu(pre), w2_ref[...],
                 preferred_element_type=jnp.float32) + b2_ref[...])
    t = jax.nn.relu(jnp.dot(m, wx1_ref[...],
                            preferred_element_type=jnp.float32) + bx1_ref[...])
    coef = jnp.sum(t * wx2_ref[...], axis=1, keepdims=True) + bx2_ref[...]
    pad = jnp.zeros((m.shape[0], PACK - H - 3), jnp.float32)
    out_ref[...] = jnp.concatenate([m, dvec * coef, pad], axis=1)


def _tc_edges(gath, et_f, mu_row, etab,
              w1s, w1d, w1r, w1w, b1, w2, b2, wx1, bx1, wx2row, bx2s):
    grid = E_C // EBLK
    dof = E_C // EBLK
    full = lambda shape: pl.BlockSpec(shape, lambda i: (0, 0))
    return pl.pallas_call(
        _edge_body,
        grid=(grid,),
        in_specs=[
            pl.BlockSpec((EBLK, GPACK), lambda i: (i, 0)),
            pl.BlockSpec((EBLK, GPACK), lambda i: (i + dof, 0)),
            pl.BlockSpec((EBLK, 1), lambda i: (i, 0)),
            full((1, NGP)), full((ED, H)),
            full((H, H)), full((H, H)), full((NGP, H)), full((H, H)),
            full((1, H)),
            full((H, H)), full((1, H)),
            full((H, H)), full((1, H)), full((1, H)), full((1, 1)),
        ],
        out_specs=pl.BlockSpec((EBLK, PACK), lambda i: (i, 0)),
        out_shape=jax.ShapeDtypeStruct((E_C, PACK), jnp.float32),
    )(gath, gath, et_f, mu_row, etab,
      w1s, w1d, w1r, w1w, b1, w2, b2, wx1, bx1, wx2row, bx2s)


def _node_body(tab_ref, p0_ref, p1_ref, p2_ref, p3_ref,
               wh1h_ref, wh1a_ref, bh1_ref,
               wh2_ref, bh2_ref, out_ref, outg_ref):
    tab = tab_ref[...]
    accum = p0_ref[0] + p1_ref[0] + p2_ref[0] + p3_ref[0]
    h = tab[:, :H]
    x = tab[:, H:H + 3]
    agg = accum[:, :H]
    dx = accum[:, H:H + 3]
    u = jax.nn.relu(
        jnp.dot(h, wh1h_ref[...], preferred_element_type=jnp.float32)
        + jnp.dot(agg, wh1a_ref[...], preferred_element_type=jnp.float32)
        + bh1_ref[...])
    hn = h + jnp.dot(u, wh2_ref[...],
                     preferred_element_type=jnp.float32) + bh2_ref[...]
    xn = x + dx
    pad = jnp.zeros((tab.shape[0], PACK - H - 3), jnp.float32)
    out_ref[...] = jnp.concatenate([hn, xn, pad], axis=1)
    outg_ref[...] = _pack_gather_rows(hn, xn)


def _tc_nodes(tab, part_a, part_b, wh1h, wh1a, bh1, wh2, bh2):
    grid = N_PAD // NBLK
    full = lambda shape: pl.BlockSpec(shape, lambda i: (0, 0))
    return pl.pallas_call(
        _node_body,
        grid=(grid,),
        in_specs=[
            pl.BlockSpec((NBLK, PACK), lambda i: (i, 0)),
            pl.BlockSpec((1, NBLK, PACK), lambda i: (0, i, 0)),
            pl.BlockSpec((1, NBLK, PACK), lambda i: (1, i, 0)),
            pl.BlockSpec((1, NBLK, PACK), lambda i: (0, i, 0)),
            pl.BlockSpec((1, NBLK, PACK), lambda i: (1, i, 0)),
            full((H, H)), full((H, H)), full((1, H)),
            full((H, H)), full((1, H)),
        ],
        out_specs=[pl.BlockSpec((NBLK, PACK), lambda i: (i, 0)),
                   pl.BlockSpec((NBLK, GPACK), lambda i: (i, 0))],
        out_shape=[jax.ShapeDtypeStruct((N_PAD, PACK), jnp.float32),
                   jax.ShapeDtypeStruct((N_PAD, GPACK), jnp.bfloat16)],
    )(tab, part_a, part_a, part_b, part_b, wh1h, wh1a, bh1, wh2, bh2)


RBLK = 1000


def _read_body(tab_ref, bat_ref, wo1_ref, bo1_ref, wo2_ref, bo2_ref, out_ref):
    i = pl.program_id(0)

    @pl.when(i == 0)
    def _():
        out_ref[...] = jnp.zeros_like(out_ref)

    h = tab_ref[...][:, :H]
    t = jax.nn.relu(jnp.dot(h, wo1_ref[...],
                            preferred_element_type=jnp.float32) + bo1_ref[...])
    ho = jnp.sum(t * wo2_ref[...], axis=1, keepdims=True) + bo2_ref[...]
    ids = jax.lax.broadcasted_iota(jnp.int32, (1, B), 1).astype(jnp.float32)
    mask = bat_ref[...] == ids
    out_ref[...] += jnp.sum(jnp.where(mask, ho, 0.0), axis=0, keepdims=True)


def _tc_readout(tab, bat_f, wo1, bo1, wo2row, bo2s):
    grid = N // RBLK
    full = lambda shape: pl.BlockSpec(shape, lambda i: (0, 0))
    return pl.pallas_call(
        _read_body,
        grid=(grid,),
        in_specs=[
            pl.BlockSpec((RBLK, PACK), lambda i: (i, 0)),
            pl.BlockSpec((RBLK, 1), lambda i: (i, 0)),
            full((H, H)), full((1, H)), full((1, H)), full((1, 1)),
        ],
        out_specs=pl.BlockSpec((1, B), lambda i: (0, 0)),
        out_shape=jax.ShapeDtypeStruct((1, B), jnp.float32),
    )(tab, bat_f, wo1, bo1, wo2row, bo2s)


def kernel(node_type, remap_node_type, pos, edge_index, edge_type, batch,
           W_node, b_node, edge_table, We1, be1, We2, be2, Wx1, bx1, Wx2, bx2,
           Wh1, bh1, Wh2, bh2, Wo1, bo1, Wo2, bo2):
    f32 = jnp.float32
    src = edge_index[0].astype(jnp.int32)
    dst = edge_index[1].astype(jnp.int32)
    padE = E_PAD - E
    src_p = jnp.concatenate([src, jnp.full((padE,), DUMMY, jnp.int32)])
    dst_p = jnp.concatenate([dst, jnp.full((padE,), DUMMY, jnp.int32)])
    gidx_ch = []
    dst_ch = []
    for k in range(CH):
        s_k = lax.dynamic_slice_in_dim(src_p, k * E_C, E_C)
        d_k = lax.dynamic_slice_in_dim(dst_p, k * E_C, E_C)
        gidx_ch.append(jnp.concatenate([s_k, d_k])
                       .reshape(NW, G_CH_W * CHUNK))
        dst_ch.append(d_k.reshape(NW, S_CH_W, SCHUNK))

    padN = N_PAD - N
    remap_pad = jnp.pad(remap_node_type.astype(f32), ((0, padN), (0, 8 - NT)))
    ntf_pad = jnp.pad(node_type.astype(f32)[:, None], ((0, padN), (0, 0)))
    pos_pad = jnp.pad(pos.astype(f32), ((0, padN), (0, 0)))
    et_f = jnp.pad(edge_type.astype(f32)[:, None], ((0, padE), (0, 0)))
    et_ch = [lax.dynamic_slice_in_dim(et_f, k * E_C, E_C) for k in range(CH)]
    bat_f = batch.astype(f32)[:, None]
    zeros_tab = jnp.zeros((N_PAD, PACK), f32)

    Wn = W_node.astype(f32).reshape(NT, 3, H)
    w0 = jnp.pad(Wn[:, 0, :], ((0, 8 - NT), (0, 0)))
    w1 = jnp.pad(Wn[:, 1, :], ((0, 8 - NT), (0, 0)))
    w2 = jnp.pad(Wn[:, 2, :], ((0, 8 - NT), (0, 0)))
    bn = b_node.astype(f32)[None, :]

    mu_row = jnp.pad(jnp.linspace(0.0, 10.0, NG), (0, NGP - NG))[None, :]
    etab = edge_table.astype(f32)

    tab, tabg = _tc_init(remap_pad, ntf_pad, pos_pad, w0, w1, w2, bn)

    for l in range(L):
        w1s = We1[l][:H].astype(f32)
        w1d = We1[l][H:2 * H].astype(f32)
        w1r = jnp.pad(We1[l][2 * H:2 * H + NG].astype(f32),
                      ((0, NGP - NG), (0, 0)))
        w1w = We1[l][2 * H + NG:].astype(f32)
        b1 = be1[l].astype(f32)[None, :]
        w2l = We2[l].astype(f32)
        b2 = be2[l].astype(f32)[None, :]
        wx1 = Wx1[l].astype(f32)
        bx1l = bx1[l].astype(f32)[None, :]
        wx2row = Wx2[l].astype(f32).reshape(1, H)
        bx2s = bx2[l].astype(f32).reshape(1, 1)
        wh1h = Wh1[l][:H].astype(f32)
        wh1a = Wh1[l][H:].astype(f32)
        bh1l = bh1[l].astype(f32)[None, :]
        wh2 = Wh2[l].astype(f32)
        bh2l = bh2[l].astype(f32)[None, :]

        parts = []
        for k in range(CH):
            gath = _sc_gather(tabg, gidx_ch[k])
            msg = _tc_edges(gath, et_ch[k], mu_row, etab,
                            w1s, w1d, w1r, w1w, b1, w2l, b2,
                            wx1, bx1l, wx2row, bx2s)
            parts.append(_sc_scatter(msg, dst_ch[k], zeros_tab))
        tab, tabg = _tc_nodes(tab, parts[0], parts[1],
                              wh1h, wh1a, bh1l, wh2, bh2l)

    wo2row = Wo2.astype(f32).reshape(1, H)
    bo2s = bo2.astype(f32).reshape(1, 1)
    out_row = _tc_readout(tab, bat_f, Wo1.astype(f32),
                          bo1.astype(f32)[None, :], wo2row, bo2s)
    out = out_row.reshape(B, 1)
    x_out = tab[:N, H:H + 3]
    return (out, x_out)


# speedup vs baseline: 1.0070x; 1.0070x over previous
import functools

import jax
import jax.numpy as jnp
from jax import lax
from jax.experimental import pallas as pl
from jax.experimental.pallas import tpu as pltpu
from jax.experimental.pallas import tpu_sc as plsc

N = 10000
E = 160000
H = 128
L = 3
NG = 20
NT = 5
ED = 4
B = 64

NGP = 24
PACK = 144
GPACK = 160
N_PAD = 10240
E_PAD = 163840
DUMMY = N

NC = 2
NS = 16
NW = NC * NS
CHUNK = 128

CH = 2
E_C = E_PAD // CH
G_ROWS = 2 * E_C
G_CH_W = G_ROWS // NW // CHUNK
SCHUNK = 64
S_NBUF = 2
S_CH_W = E_C // NW // SCHUNK
ROWS_SUB = N_PAD // NS

EBLK = 1024
NBLK = 1024

def _mesh():
    return plsc.VectorSubcoreMesh(core_axis_name="c", subcore_axis_name="s")


NBUF = 4


GBUF = 8


def _sc_gather(table, idx2):
    @functools.partial(
        pl.kernel,
        out_type=jax.ShapeDtypeStruct((G_ROWS, GPACK), jnp.bfloat16),
        mesh=_mesh(),
        compiler_params=pltpu.CompilerParams(use_tc_tiling_on_sc=False),
        scratch_types=[
            pltpu.VMEM((G_CH_W * CHUNK,), jnp.int32),
        ] + [pltpu.VMEM((CHUNK, GPACK), jnp.bfloat16)] * GBUF
          + [pltpu.SemaphoreType.DMA] * (2 * GBUF),
    )
    def k(table_hbm, idx_hbm, out_hbm, idx_all, *rest):
        bufs = rest[:GBUF]
        gs = rest[GBUF:2 * GBUF]
        ws = rest[2 * GBUF:]
        wid = lax.axis_index("s") * NC + lax.axis_index("c")
        pltpu.sync_copy(idx_hbm.at[wid], idx_all)
        base_row = wid * G_CH_W * CHUNK

        def gidx(i):
            return idx_all.at[pl.ds(i * CHUNK, CHUNK)]

        def orow(i):
            return out_hbm.at[pl.ds(base_row + i * CHUNK, CHUNK)]

        def start_g(i, b):
            pltpu.async_copy(table_hbm.at[gidx(i)], bufs[b], gs[b])

        def wait_g(i, b):
            pltpu.make_async_copy(table_hbm.at[gidx(i)], bufs[b],
                                  gs[b]).wait()

        def start_w(i, b):
            pltpu.async_copy(bufs[b], orow(i), ws[b])

        def wait_w(i, b):
            pltpu.make_async_copy(bufs[b], orow(i), ws[b]).wait()

        for b in range(GBUF):
            start_g(b, b)

        @pl.loop(0, G_CH_W // GBUF - 1)
        def _(j):
            for b in range(GBUF):
                wait_g(j * GBUF + b, b)
                start_w(j * GBUF + b, b)
            for b in range(GBUF):
                wait_w(j * GBUF + b, b)
                start_g((j + 1) * GBUF + b, b)

        last = G_CH_W - GBUF
        for b in range(GBUF):
            wait_g(last + b, b)
            start_w(last + b, b)
        for b in range(GBUF):
            wait_w(last + b, b)

    return k(table, idx2)


def _sc_scatter(msg, dst3, zeros_tab):
    @functools.partial(
        pl.kernel,
        out_type=jax.ShapeDtypeStruct((NC, N_PAD, PACK), jnp.float32),
        mesh=_mesh(),
        compiler_params=pltpu.CompilerParams(use_tc_tiling_on_sc=False),
        scratch_types=[
            pltpu.VMEM((S_CH_W, SCHUNK), jnp.int32),
        ] + [pltpu.VMEM((SCHUNK, PACK), jnp.float32)] * S_NBUF
          + [pltpu.VMEM_SHARED((N_PAD, PACK), jnp.float32)]
          + [pltpu.SemaphoreType.DMA] * S_NBUF,
    )
    def k(msg_hbm, dst_hbm, zeros_hbm, out_hbm, idx_all, b0, b1,
          acc_sh, s0, s1):
        bufs = (b0, b1)
        sems = (s0, s1)
        c = lax.axis_index("c")
        s = lax.axis_index("s")
        wid = s * NC + c
        pltpu.sync_copy(zeros_hbm.at[pl.ds(s * ROWS_SUB, ROWS_SUB)],
                        acc_sh.at[pl.ds(s * ROWS_SUB, ROWS_SUB)])
        pltpu.sync_copy(dst_hbm.at[wid], idx_all)
        plsc.subcore_barrier()
        base_e = wid * S_CH_W * SCHUNK

        def mrow(i):
            return msg_hbm.at[pl.ds(base_e + i * SCHUNK, SCHUNK)]

        for b in range(S_NBUF):
            pltpu.async_copy(mrow(b), bufs[b], sems[b])

        def step(i, b):
            pltpu.make_async_copy(mrow(i), bufs[b], sems[b]).wait()
            pltpu.sync_copy(bufs[b], acc_sh.at[idx_all.at[i]], add=True)

        @pl.loop(0, S_CH_W // S_NBUF - 1)
        def _(j):
            for b in range(S_NBUF):
                i = j * S_NBUF + b
                step(i, b)
                pltpu.async_copy(mrow(i + S_NBUF), bufs[b], sems[b])

        for b in range(S_NBUF):
            step(S_CH_W - S_NBUF + b, b)

        plsc.subcore_barrier()
        pltpu.sync_copy(acc_sh.at[pl.ds(s * ROWS_SUB, ROWS_SUB)],
                        out_hbm.at[c].at[pl.ds(s * ROWS_SUB, ROWS_SUB)])

    return k(msg, dst3, zeros_tab)


def _pack_gather_rows(h, x):
    bf = jnp.bfloat16
    x_hi = x.astype(bf)
    x_lo = (x - x_hi.astype(jnp.float32)).astype(bf)
    pad = jnp.zeros((h.shape[0], GPACK - H - 6), bf)
    return jnp.concatenate([h.astype(bf), x_hi, x_lo, pad], axis=1)


def _init_body(remap_ref, ntf_ref, pos_ref, w0_ref, w1_ref, w2_ref, b_ref,
               out_ref, outg_ref):
    c = ntf_ref[...] / 9.0
    remap = remap_ref[...]
    h = (jnp.dot(remap, w0_ref[...], preferred_element_type=jnp.float32)
         + jnp.dot(remap * c, w1_ref[...], preferred_element_type=jnp.float32)
         + jnp.dot(remap * (c * c), w2_ref[...],
                   preferred_element_type=jnp.float32)
         + b_ref[...])
    pos = pos_ref[...]
    pad = jnp.zeros((out_ref.shape[0], PACK - H - 3), jnp.float32)
    out_ref[...] = jnp.concatenate([h, pos, pad], axis=1)
    outg_ref[...] = _pack_gather_rows(h, pos)


def _tc_init(remap_pad, ntf_pad, pos_pad, w0, w1, w2, b_node):
    grid = N_PAD // NBLK
    full = lambda shape: pl.BlockSpec(shape, lambda i: (0, 0))
    return pl.pallas_call(
        _init_body,
        grid=(grid,),
        in_specs=[
            pl.BlockSpec((NBLK, 8), lambda i: (i, 0)),
            pl.BlockSpec((NBLK, 1), lambda i: (i, 0)),
            pl.BlockSpec((NBLK, 3), lambda i: (i, 0)),
            full((8, H)), full((8, H)), full((8, H)), full((1, H)),
        ],
        out_specs=[pl.BlockSpec((NBLK, PACK), lambda i: (i, 0)),
                   pl.BlockSpec((NBLK, GPACK), lambda i: (i, 0))],
        out_shape=[jax.ShapeDtypeStruct((N_PAD, PACK), jnp.float32),
                   jax.ShapeDtypeStruct((N_PAD, GPACK), jnp.bfloat16)],
    )(remap_pad, ntf_pad, pos_pad, w0, w1, w2, b_node)


def _edge_body(gs_ref, gd_ref, et_ref, mu_ref, etw_ref,
               w1s_ref, w1d_ref, w1r_ref, b1_ref,
               w2_ref, b2_ref, wx1_ref, bx1_ref, wx2_ref, bx2_ref,
               out_ref):
    gs = gs_ref[...]
    gd = gd_ref[...]
    f32 = jnp.float32
    bf = jnp.bfloat16
    hs = gs[:, :H]
    hd = gd[:, :H]
    xs = gs[:, H:H + 3].astype(f32) + gs[:, H + 3:H + 6].astype(f32)
    xd = gd[:, H:H + 3].astype(f32) + gd[:, H + 3:H + 6].astype(f32)
    dvec = xd - xs
    d = jnp.sqrt(jnp.sum(dvec * dvec, axis=1, keepdims=True) + 1e-8)
    sigma = 10.0 / NG
    rbf = jnp.exp(-((d - mu_ref[...]) ** 2) / (2.0 * sigma * sigma))

    et = et_ref[...]
    w = jnp.zeros((et.shape[0], H), jnp.float32)
    for kk in range(ED):
        w = w + jnp.where(et == float(kk), 1.0, 0.0) * etw_ref[kk:kk + 1, :]

    pre = (jnp.dot(hs, w1s_ref[...], preferred_element_type=f32)
           + jnp.dot(hd, w1d_ref[...], preferred_element_type=f32)
           + jnp.dot(rbf.astype(bf), w1r_ref[...], preferred_element_type=f32)
           + w + b1_ref[...])
    m = (jnp.dot(jax.nn.relu(pre).astype(bf), w2_ref[...],
                 preferred_element_type=f32) + b2_ref[...])
    t = jax.nn.relu(jnp.dot(m.astype(bf), wx1_ref[...],
                            preferred_element_type=f32) + bx1_ref[...])
    coef = jnp.sum(t * wx2_ref[...], axis=1, keepdims=True) + bx2_ref[...]
    pad = jnp.zeros((m.shape[0], PACK - H - 3), jnp.float32)
    out_ref[...] = jnp.concatenate([m, dvec * coef, pad], axis=1)


def _tc_edges(gath, et_f, mu_row, etw,
              w1s, w1d, w1r, b1, w2, b2, wx1, bx1, wx2row, bx2s):
    grid = E_C // EBLK
    dof = E_C // EBLK
    full = lambda shape: pl.BlockSpec(shape, lambda i: (0, 0))
    return pl.pallas_call(
        _edge_body,
        grid=(grid,),
        in_specs=[
            pl.BlockSpec((EBLK, GPACK), lambda i: (i, 0)),
            pl.BlockSpec((EBLK, GPACK), lambda i: (i + dof, 0)),
            pl.BlockSpec((EBLK, 1), lambda i: (i, 0)),
            full((1, NGP)), full((ED, H)),
            full((H, H)), full((H, H)), full((NGP, H)),
            full((1, H)),
            full((H, H)), full((1, H)),
            full((H, H)), full((1, H)), full((1, H)), full((1, 1)),
        ],
        out_specs=pl.BlockSpec((EBLK, PACK), lambda i: (i, 0)),
        out_shape=jax.ShapeDtypeStruct((E_C, PACK), jnp.float32),
    )(gath, gath, et_f, mu_row, etw,
      w1s, w1d, w1r, b1, w2, b2, wx1, bx1, wx2row, bx2s)


def _etw_body(etab_ref, w1w_ref, out_ref):
    out_ref[...] = jnp.dot(etab_ref[...], w1w_ref[0],
                           preferred_element_type=jnp.float32)[None]


def _tc_etw(etab, w1w_stack):
    full = lambda shape: pl.BlockSpec(shape, lambda i: (0, 0))
    return pl.pallas_call(
        _etw_body,
        grid=(L,),
        in_specs=[full((ED, H)),
                  pl.BlockSpec((1, H, H), lambda i: (i, 0, 0))],
        out_specs=pl.BlockSpec((1, ED, H), lambda i: (i, 0, 0)),
        out_shape=jax.ShapeDtypeStruct((L, ED, H), jnp.float32),
    )(etab, w1w_stack)


def _node_body(tab_ref, p0_ref, p1_ref, p2_ref, p3_ref,
               wh1h_ref, wh1a_ref, bh1_ref,
               wh2_ref, bh2_ref, out_ref, outg_ref):
    tab = tab_ref[...]
    accum = p0_ref[0] + p1_ref[0] + p2_ref[0] + p3_ref[0]
    h = tab[:, :H]
    x = tab[:, H:H + 3]
    agg = accum[:, :H]
    dx = accum[:, H:H + 3]
    u = jax.nn.relu(
        jnp.dot(h, wh1h_ref[...], preferred_element_type=jnp.float32)
        + jnp.dot(agg, wh1a_ref[...], preferred_element_type=jnp.float32)
        + bh1_ref[...])
    hn = h + jnp.dot(u, wh2_ref[...],
                     preferred_element_type=jnp.float32) + bh2_ref[...]
    xn = x + dx
    pad = jnp.zeros((tab.shape[0], PACK - H - 3), jnp.float32)
    out_ref[...] = jnp.concatenate([hn, xn, pad], axis=1)
    outg_ref[...] = _pack_gather_rows(hn, xn)


def _tc_nodes(tab, part_a, part_b, wh1h, wh1a, bh1, wh2, bh2):
    grid = N_PAD // NBLK
    full = lambda shape: pl.BlockSpec(shape, lambda i: (0, 0))
    return pl.pallas_call(
        _node_body,
        grid=(grid,),
        in_specs=[
            pl.BlockSpec((NBLK, PACK), lambda i: (i, 0)),
            pl.BlockSpec((1, NBLK, PACK), lambda i: (0, i, 0)),
            pl.BlockSpec((1, NBLK, PACK), lambda i: (1, i, 0)),
            pl.BlockSpec((1, NBLK, PACK), lambda i: (0, i, 0)),
            pl.BlockSpec((1, NBLK, PACK), lambda i: (1, i, 0)),
            full((H, H)), full((H, H)), full((1, H)),
            full((H, H)), full((1, H)),
        ],
        out_specs=[pl.BlockSpec((NBLK, PACK), lambda i: (i, 0)),
                   pl.BlockSpec((NBLK, GPACK), lambda i: (i, 0))],
        out_shape=[jax.ShapeDtypeStruct((N_PAD, PACK), jnp.float32),
                   jax.ShapeDtypeStruct((N_PAD, GPACK), jnp.bfloat16)],
    )(tab, part_a, part_a, part_b, part_b, wh1h, wh1a, bh1, wh2, bh2)


RBLK = 1000


def _read_body(tab_ref, bat_ref, wo1_ref, bo1_ref, wo2_ref, bo2_ref, out_ref):
    i = pl.program_id(0)

    @pl.when(i == 0)
    def _():
        out_ref[...] = jnp.zeros_like(out_ref)

    h = tab_ref[...][:, :H]
    t = jax.nn.relu(jnp.dot(h, wo1_ref[...],
                            preferred_element_type=jnp.float32) + bo1_ref[...])
    ho = jnp.sum(t * wo2_ref[...], axis=1, keepdims=True) + bo2_ref[...]
    ids = jax.lax.broadcasted_iota(jnp.int32, (1, B), 1).astype(jnp.float32)
    mask = bat_ref[...] == ids
    out_ref[...] += jnp.sum(jnp.where(mask, ho, 0.0), axis=0, keepdims=True)


def _tc_readout(tab, bat_f, wo1, bo1, wo2row, bo2s):
    grid = N // RBLK
    full = lambda shape: pl.BlockSpec(shape, lambda i: (0, 0))
    return pl.pallas_call(
        _read_body,
        grid=(grid,),
        in_specs=[
            pl.BlockSpec((RBLK, PACK), lambda i: (i, 0)),
            pl.BlockSpec((RBLK, 1), lambda i: (i, 0)),
            full((H, H)), full((1, H)), full((1, H)), full((1, 1)),
        ],
        out_specs=pl.BlockSpec((1, B), lambda i: (0, 0)),
        out_shape=jax.ShapeDtypeStruct((1, B), jnp.float32),
    )(tab, bat_f, wo1, bo1, wo2row, bo2s)


def kernel(node_type, remap_node_type, pos, edge_index, edge_type, batch,
           W_node, b_node, edge_table, We1, be1, We2, be2, Wx1, bx1, Wx2, bx2,
           Wh1, bh1, Wh2, bh2, Wo1, bo1, Wo2, bo2):
    f32 = jnp.float32
    src = edge_index[0].astype(jnp.int32)
    dst = edge_index[1].astype(jnp.int32)
    padE = E_PAD - E
    src_p = jnp.concatenate([src, jnp.full((padE,), DUMMY, jnp.int32)])
    dst_p = jnp.concatenate([dst, jnp.full((padE,), DUMMY, jnp.int32)])
    gidx_ch = []
    dst_ch = []
    for k in range(CH):
        s_k = lax.dynamic_slice_in_dim(src_p, k * E_C, E_C)
        d_k = lax.dynamic_slice_in_dim(dst_p, k * E_C, E_C)
        gidx_ch.append(jnp.concatenate([s_k, d_k])
                       .reshape(NW, G_CH_W * CHUNK))
        dst_ch.append(d_k.reshape(NW, S_CH_W, SCHUNK))

    padN = N_PAD - N
    remap_pad = jnp.pad(remap_node_type.astype(f32), ((0, padN), (0, 8 - NT)))
    ntf_pad = jnp.pad(node_type.astype(f32)[:, None], ((0, padN), (0, 0)))
    pos_pad = jnp.pad(pos.astype(f32), ((0, padN), (0, 0)))
    et_f = jnp.pad(edge_type.astype(f32)[:, None], ((0, padE), (0, 0)))
    et_ch = [lax.dynamic_slice_in_dim(et_f, k * E_C, E_C) for k in range(CH)]
    bat_f = batch.astype(f32)[:, None]
    zeros_tab = jnp.zeros((N_PAD, PACK), f32)

    Wn = W_node.astype(f32).reshape(NT, 3, H)
    w0 = jnp.pad(Wn[:, 0, :], ((0, 8 - NT), (0, 0)))
    w1 = jnp.pad(Wn[:, 1, :], ((0, 8 - NT), (0, 0)))
    w2 = jnp.pad(Wn[:, 2, :], ((0, 8 - NT), (0, 0)))
    bn = b_node.astype(f32)[None, :]

    mu_row = jnp.pad(jnp.linspace(0.0, 10.0, NG), (0, NGP - NG))[None, :]
    etab = edge_table.astype(f32)
    w1w_stack = jnp.stack([We1[l][2 * H + NG:].astype(f32)
                           for l in range(L)])
    etw_all = _tc_etw(etab, w1w_stack)

    tab, tabg = _tc_init(remap_pad, ntf_pad, pos_pad, w0, w1, w2, bn)

    bf = jnp.bfloat16
    for l in range(L):
        w1s = We1[l][:H].astype(bf)
        w1d = We1[l][H:2 * H].astype(bf)
        w1r = jnp.pad(We1[l][2 * H:2 * H + NG].astype(bf),
                      ((0, NGP - NG), (0, 0)))
        etw = etw_all[l]
        b1 = be1[l].astype(f32)[None, :]
        w2l = We2[l].astype(bf)
        b2 = be2[l].astype(f32)[None, :]
        wx1 = Wx1[l].astype(bf)
        bx1l = bx1[l].astype(f32)[None, :]
        wx2row = Wx2[l].astype(f32).reshape(1, H)
        bx2s = bx2[l].astype(f32).reshape(1, 1)
        wh1h = Wh1[l][:H].astype(f32)
        wh1a = Wh1[l][H:].astype(f32)
        bh1l = bh1[l].astype(f32)[None, :]
        wh2 = Wh2[l].astype(f32)
        bh2l = bh2[l].astype(f32)[None, :]

        parts = []
        for k in range(CH):
            gath = _sc_gather(tabg, gidx_ch[k])
            msg = _tc_edges(gath, et_ch[k], mu_row, etw,
                            w1s, w1d, w1r, b1, w2l, b2,
                            wx1, bx1l, wx2row, bx2s)
            parts.append(_sc_scatter(msg, dst_ch[k], zeros_tab))
        tab, tabg = _tc_nodes(tab, parts[0], parts[1],
                              wh1h, wh1a, bh1l, wh2, bh2l)

    wo2row = Wo2.astype(f32).reshape(1, H)
    bo2s = bo2.astype(f32).reshape(1, 1)
    out_row = _tc_readout(tab, bat_f, Wo1.astype(f32),
                          bo1.astype(f32)[None, :], wo2row, bo2s)
    out = out_row.reshape(B, 1)
    x_out = tab[:N, H:H + 3]
    return (out, x_out)


# speedup vs baseline: 1.4305x; 1.4206x over previous
import functools

import jax
import jax.numpy as jnp
from jax import lax
from jax.experimental import pallas as pl
from jax.experimental.pallas import tpu as pltpu
from jax.experimental.pallas import tpu_sc as plsc

N = 10000
E = 160000
H = 128
L = 3
NG = 20
NT = 5
ED = 4
B = 64

NGP = 24
PACK = 144
GW = 128
XW = 16
N_PAD = 10240
E_PAD = 163840
DUMMY = N

NC = 2
NS = 16
NW = NC * NS
CHUNK = 128

CH = 2
E_C = E_PAD // CH
G_ROWS = 2 * E_C
G_CH_W = G_ROWS // NW // CHUNK
SCHUNK = 64
S_NBUF = 2
S_CH_W = E_C // NW // SCHUNK
ROWS_SUB = N_PAD // NS

EBLK = 1024
NBLK = 1024

GBUF = 4


def _mesh():
    return plsc.VectorSubcoreMesh(core_axis_name="c", subcore_axis_name="s")


def _u32(v):
    return jax.lax.bitcast_convert_type(v, jnp.uint32)


def _f32(v):
    return jax.lax.bitcast_convert_type(v, jnp.float32)


def _pack_gather_words(h, x):
    r = jnp.uint32(0x8000)
    we = (_u32(h[:, :64]) + r) >> 16
    wo = (_u32(h[:, 64:]) + r) & jnp.uint32(0xffff0000)
    hw = we | wo
    xhi_bits = (_u32(x) + r) >> 16
    xhi = _f32(xhi_bits << 16)
    xlo = x - xhi
    xlo_bits = (_u32(xlo) + r) & jnp.uint32(0xffff0000)
    xw = xhi_bits | xlo_bits
    pad = jnp.zeros((h.shape[0], GW - 64 - 3), jnp.uint32)
    return jnp.concatenate([hw, xw, pad], axis=1)


def _unpack_gather_words(w):
    lo = _f32(w << 16)
    hi = _f32(w & jnp.uint32(0xffff0000))
    h = jnp.concatenate([lo[:, :64], hi[:, :64]], axis=1)
    x = lo[:, 64:67] + hi[:, 64:67]
    return h, x


def _sc_gather(table, idx2):
    @functools.partial(
        pl.kernel,
        out_type=jax.ShapeDtypeStruct((G_ROWS, GW), jnp.uint32),
        mesh=_mesh(),
        compiler_params=pltpu.CompilerParams(use_tc_tiling_on_sc=False),
        scratch_types=[
            pltpu.VMEM((G_CH_W * CHUNK,), jnp.int32),
        ] + [pltpu.VMEM((CHUNK, GW), jnp.uint32)] * GBUF
          + [pltpu.SemaphoreType.DMA] * (2 * GBUF),
    )
    def k(table_hbm, idx_hbm, out_hbm, idx_all, *rest):
        bufs = rest[:GBUF]
        gs = rest[GBUF:2 * GBUF]
        ws = rest[2 * GBUF:]
        wid = lax.axis_index("s") * NC + lax.axis_index("c")
        pltpu.sync_copy(idx_hbm.at[wid], idx_all)
        base_row = wid * G_CH_W * CHUNK

        def gidx(i):
            return idx_all.at[pl.ds(i * CHUNK, CHUNK)]

        def orow(i):
            return out_hbm.at[pl.ds(base_row + i * CHUNK, CHUNK)]

        def start_g(i, b):
            pltpu.async_copy(table_hbm.at[gidx(i)], bufs[b], gs[b])

        def wait_g(i, b):
            pltpu.make_async_copy(table_hbm.at[gidx(i)], bufs[b],
                                  gs[b]).wait()

        def start_w(i, b):
            pltpu.async_copy(bufs[b], orow(i), ws[b])

        def wait_w(i, b):
            pltpu.make_async_copy(bufs[b], orow(i), ws[b]).wait()

        for b in range(GBUF):
            start_g(b, b)

        @pl.loop(0, G_CH_W // GBUF - 1)
        def _(j):
            for b in range(GBUF):
                wait_g(j * GBUF + b, b)
                start_w(j * GBUF + b, b)
            for b in range(GBUF):
                wait_w(j * GBUF + b, b)
                start_g((j + 1) * GBUF + b, b)

        last = G_CH_W - GBUF
        for b in range(GBUF):
            wait_g(last + b, b)
            start_w(last + b, b)
        for b in range(GBUF):
            wait_w(last + b, b)

    return k(table, idx2)


def _sc_scatter(msg, dst3, zeros_tab):
    @functools.partial(
        pl.kernel,
        out_type=jax.ShapeDtypeStruct((NC, N_PAD, PACK), jnp.float32),
        mesh=_mesh(),
        compiler_params=pltpu.CompilerParams(use_tc_tiling_on_sc=False),
        scratch_types=[
            pltpu.VMEM((S_CH_W, SCHUNK), jnp.int32),
        ] + [pltpu.VMEM((SCHUNK, PACK), jnp.float32)] * S_NBUF
          + [pltpu.VMEM_SHARED((N_PAD, PACK), jnp.float32)]
          + [pltpu.SemaphoreType.DMA] * S_NBUF,
    )
    def k(msg_hbm, dst_hbm, zeros_hbm, out_hbm, idx_all, b0, b1,
          acc_sh, s0, s1):
        bufs = (b0, b1)
        sems = (s0, s1)
        c = lax.axis_index("c")
        s = lax.axis_index("s")
        wid = s * NC + c
        rows = pl.ds(s * ROWS_SUB, ROWS_SUB)
        pltpu.sync_copy(zeros_hbm.at[rows], acc_sh.at[rows])
        pltpu.sync_copy(dst_hbm.at[wid], idx_all)
        plsc.subcore_barrier()
        base_e = wid * S_CH_W * SCHUNK

        def mrow(i):
            return msg_hbm.at[pl.ds(base_e + i * SCHUNK, SCHUNK)]

        for b in range(S_NBUF):
            pltpu.async_copy(mrow(b), bufs[b], sems[b])

        def step(i, b):
            pltpu.make_async_copy(mrow(i), bufs[b], sems[b]).wait()
            pltpu.sync_copy(bufs[b], acc_sh.at[idx_all.at[i]], add=True)

        @pl.loop(0, S_CH_W // S_NBUF - 1)
        def _(j):
            for b in range(S_NBUF):
                i = j * S_NBUF + b
                step(i, b)
                pltpu.async_copy(mrow(i + S_NBUF), bufs[b], sems[b])

        for b in range(S_NBUF):
            step(S_CH_W - S_NBUF + b, b)

        plsc.subcore_barrier()
        pltpu.sync_copy(acc_sh.at[rows], out_hbm.at[c].at[rows])

    return k(msg, dst3, zeros_tab)


def _init_body(remap_ref, ntf_ref, pos_ref, w0_ref, w1_ref, w2_ref, b_ref,
               out_ref, outg_ref):
    c = ntf_ref[...] / 9.0
    remap = remap_ref[...]
    h = (jnp.dot(remap, w0_ref[...], preferred_element_type=jnp.float32)
         + jnp.dot(remap * c, w1_ref[...], preferred_element_type=jnp.float32)
         + jnp.dot(remap * (c * c), w2_ref[...],
                   preferred_element_type=jnp.float32)
         + b_ref[...])
    pos = pos_ref[...]
    pad = jnp.zeros((out_ref.shape[0], PACK - H - 3), jnp.float32)
    out_ref[...] = jnp.concatenate([h, pos, pad], axis=1)
    outg_ref[...] = _pack_gather_words(h, pos)


def _tc_init(remap_pad, ntf_pad, pos_pad, w0, w1, w2, b_node):
    grid = N_PAD // NBLK
    full = lambda shape: pl.BlockSpec(shape, lambda i: (0, 0))
    return pl.pallas_call(
        _init_body,
        grid=(grid,),
        in_specs=[
            pl.BlockSpec((NBLK, 8), lambda i: (i, 0)),
            pl.BlockSpec((NBLK, 1), lambda i: (i, 0)),
            pl.BlockSpec((NBLK, 3), lambda i: (i, 0)),
            full((8, H)), full((8, H)), full((8, H)), full((1, H)),
        ],
        out_specs=[pl.BlockSpec((NBLK, PACK), lambda i: (i, 0)),
                   pl.BlockSpec((NBLK, GW), lambda i: (i, 0))],
        out_shape=[jax.ShapeDtypeStruct((N_PAD, PACK), jnp.float32),
                   jax.ShapeDtypeStruct((N_PAD, GW), jnp.uint32)],
    )(remap_pad, ntf_pad, pos_pad, w0, w1, w2, b_node)


def _edge_body(gs_ref, gd_ref, et_ref, mu_ref, etw_ref,
               w1s_ref, w1d_ref, w1r_ref, b1_ref,
               w2_ref, b2_ref, wx1_ref, bx1_ref, wx2_ref, bx2_ref,
               m_ref):
    f32 = jnp.float32
    hs, xs = _unpack_gather_words(gs_ref[...])
    hd, xd = _unpack_gather_words(gd_ref[...])
    dvec = xd - xs
    d = jnp.sqrt(jnp.sum(dvec * dvec, axis=1, keepdims=True) + 1e-8)
    sigma = 10.0 / NG
    rbf = jnp.exp(-((d - mu_ref[...]) ** 2) / (2.0 * sigma * sigma))

    et = et_ref[...]
    w = jnp.zeros((et.shape[0], H), jnp.float32)
    for kk in range(ED):
        w = w + jnp.where(et == float(kk), 1.0, 0.0) * etw_ref[kk:kk + 1, :]

    pre = (jnp.dot(hs, w1s_ref[...], preferred_element_type=f32)
           + jnp.dot(hd, w1d_ref[...], preferred_element_type=f32)
           + jnp.dot(rbf, w1r_ref[...], preferred_element_type=f32)
           + w + b1_ref[...])
    m = (jnp.dot(jax.nn.relu(pre), w2_ref[...],
                 preferred_element_type=f32) + b2_ref[...])
    t = jax.nn.relu(jnp.dot(m, wx1_ref[...],
                            preferred_element_type=f32) + bx1_ref[...])
    coef = jnp.sum(t * wx2_ref[...], axis=1, keepdims=True) + bx2_ref[...]
    pad = jnp.zeros((m.shape[0], PACK - H - 3), jnp.float32)
    m_ref[...] = jnp.concatenate([m, dvec * coef, pad], axis=1)


def _tc_edges(gath, et_f, mu_row, etw,
              w1s, w1d, w1r, b1, w2, b2, wx1, bx1, wx2row, bx2s):
    grid = E_C // EBLK
    dof = E_C // EBLK
    full = lambda shape: pl.BlockSpec(shape, lambda i: (0, 0))
    return pl.pallas_call(
        _edge_body,
        grid=(grid,),
        in_specs=[
            pl.BlockSpec((EBLK, GW), lambda i: (i, 0)),
            pl.BlockSpec((EBLK, GW), lambda i: (i + dof, 0)),
            pl.BlockSpec((EBLK, 1), lambda i: (i, 0)),
            full((1, NGP)), full((ED, H)),
            full((H, H)), full((H, H)), full((NGP, H)),
            full((1, H)),
            full((H, H)), full((1, H)),
            full((H, H)), full((1, H)), full((1, H)), full((1, 1)),
        ],
        out_specs=pl.BlockSpec((EBLK, PACK), lambda i: (i, 0)),
        out_shape=jax.ShapeDtypeStruct((E_C, PACK), jnp.float32),
    )(gath, gath, et_f, mu_row, etw,
      w1s, w1d, w1r, b1, w2, b2, wx1, bx1, wx2row, bx2s)


def _etw_body(etab_ref, w1w_ref, out_ref):
    out_ref[...] = jnp.dot(etab_ref[...], w1w_ref[0],
                           preferred_element_type=jnp.float32)[None]


def _tc_etw(etab, w1w_stack):
    full = lambda shape: pl.BlockSpec(shape, lambda i: (0, 0))
    return pl.pallas_call(
        _etw_body,
        grid=(L,),
        in_specs=[full((ED, H)),
                  pl.BlockSpec((1, H, H), lambda i: (i, 0, 0))],
        out_specs=pl.BlockSpec((1, ED, H), lambda i: (i, 0, 0)),
        out_shape=jax.ShapeDtypeStruct((L, ED, H), jnp.float32),
    )(etab, w1w_stack)


def _node_body(tab_ref, p00_ref, p01_ref, p10_ref, p11_ref,
               wh1h_ref, wh1a_ref, bh1_ref,
               wh2_ref, bh2_ref, out_ref, outg_ref):
    tab = tab_ref[...]
    aggf = (p00_ref[0] + p01_ref[0] + p10_ref[0]
            + p11_ref[0])
    agg = aggf[:, :H]
    h = tab[:, :H]
    x = tab[:, H:H + 3]
    dx = aggf[:, H:H + 3]
    u = jax.nn.relu(
        jnp.dot(h, wh1h_ref[...], preferred_element_type=jnp.float32)
        + jnp.dot(agg, wh1a_ref[...], preferred_element_type=jnp.float32)
        + bh1_ref[...])
    hn = h + jnp.dot(u, wh2_ref[...],
                     preferred_element_type=jnp.float32) + bh2_ref[...]
    xn = x + dx
    pad = jnp.zeros((tab.shape[0], PACK - H - 3), jnp.float32)
    out_ref[...] = jnp.concatenate([hn, xn, pad], axis=1)
    outg_ref[...] = _pack_gather_words(hn, xn)


def _tc_nodes(tab, part0, part1, wh1h, wh1a, bh1, wh2, bh2):
    grid = N_PAD // NBLK
    full = lambda shape: pl.BlockSpec(shape, lambda i: (0, 0))

    def slab(c):
        return pl.BlockSpec((1, NBLK, PACK), lambda i, c=c: (c, i, 0))

    return pl.pallas_call(
        _node_body,
        grid=(grid,),
        in_specs=[
            pl.BlockSpec((NBLK, PACK), lambda i: (i, 0)),
            slab(0), slab(1), slab(0), slab(1),
            full((H, H)), full((H, H)), full((1, H)),
            full((H, H)), full((1, H)),
        ],
        out_specs=[pl.BlockSpec((NBLK, PACK), lambda i: (i, 0)),
                   pl.BlockSpec((NBLK, GW), lambda i: (i, 0))],
        out_shape=[jax.ShapeDtypeStruct((N_PAD, PACK), jnp.float32),
                   jax.ShapeDtypeStruct((N_PAD, GW), jnp.uint32)],
    )(tab, part0, part0, part1, part1,
      wh1h, wh1a, bh1, wh2, bh2)


RBLK = 1000


def _read_body(tab_ref, bat_ref, wo1_ref, bo1_ref, wo2_ref, bo2_ref, out_ref):
    i = pl.program_id(0)

    @pl.when(i == 0)
    def _():
        out_ref[...] = jnp.zeros_like(out_ref)

    h = tab_ref[...][:, :H]
    t = jax.nn.relu(jnp.dot(h, wo1_ref[...],
                            preferred_element_type=jnp.float32) + bo1_ref[...])
    ho = jnp.sum(t * wo2_ref[...], axis=1, keepdims=True) + bo2_ref[...]
    ids = jax.lax.broadcasted_iota(jnp.int32, (1, B), 1).astype(jnp.float32)
    mask = bat_ref[...] == ids
    out_ref[...] += jnp.sum(jnp.where(mask, ho, 0.0), axis=0, keepdims=True)


def _tc_readout(tab, bat_f, wo1, bo1, wo2row, bo2s):
    grid = N // RBLK
    full = lambda shape: pl.BlockSpec(shape, lambda i: (0, 0))
    return pl.pallas_call(
        _read_body,
        grid=(grid,),
        in_specs=[
            pl.BlockSpec((RBLK, PACK), lambda i: (i, 0)),
            pl.BlockSpec((RBLK, 1), lambda i: (i, 0)),
            full((H, H)), full((1, H)), full((1, H)), full((1, 1)),
        ],
        out_specs=pl.BlockSpec((1, B), lambda i: (0, 0)),
        out_shape=jax.ShapeDtypeStruct((1, B), jnp.float32),
    )(tab, bat_f, wo1, bo1, wo2row, bo2s)


def kernel(node_type, remap_node_type, pos, edge_index, edge_type, batch,
           W_node, b_node, edge_table, We1, be1, We2, be2, Wx1, bx1, Wx2, bx2,
           Wh1, bh1, Wh2, bh2, Wo1, bo1, Wo2, bo2):
    f32 = jnp.float32
    src = edge_index[0].astype(jnp.int32)
    dst = edge_index[1].astype(jnp.int32)
    padE = E_PAD - E
    src_p = jnp.concatenate([src, jnp.full((padE,), DUMMY, jnp.int32)])
    dst_p = jnp.concatenate([dst, jnp.full((padE,), DUMMY, jnp.int32)])
    gidx_ch = []
    dst_ch = []
    for k in range(CH):
        s_k = lax.dynamic_slice_in_dim(src_p, k * E_C, E_C)
        d_k = lax.dynamic_slice_in_dim(dst_p, k * E_C, E_C)
        gidx_ch.append(jnp.concatenate([s_k, d_k])
                       .reshape(NW, G_CH_W * CHUNK))
        dst_ch.append(d_k.reshape(NW, S_CH_W, SCHUNK))

    padN = N_PAD - N
    remap_pad = jnp.pad(remap_node_type.astype(f32), ((0, padN), (0, 8 - NT)))
    ntf_pad = jnp.pad(node_type.astype(f32)[:, None], ((0, padN), (0, 0)))
    pos_pad = jnp.pad(pos.astype(f32), ((0, padN), (0, 0)))
    et_f = jnp.pad(edge_type.astype(f32)[:, None], ((0, padE), (0, 0)))
    et_ch = [lax.dynamic_slice_in_dim(et_f, k * E_C, E_C) for k in range(CH)]
    bat_f = batch.astype(f32)[:, None]
    zeros_pack = jnp.zeros((N_PAD, PACK), f32)

    Wn = W_node.astype(f32).reshape(NT, 3, H)
    w0 = jnp.pad(Wn[:, 0, :], ((0, 8 - NT), (0, 0)))
    w1 = jnp.pad(Wn[:, 1, :], ((0, 8 - NT), (0, 0)))
    w2 = jnp.pad(Wn[:, 2, :], ((0, 8 - NT), (0, 0)))
    bn = b_node.astype(f32)[None, :]

    mu_row = jnp.pad(jnp.linspace(0.0, 10.0, NG), (0, NGP - NG))[None, :]
    etab = edge_table.astype(f32)
    w1w_stack = jnp.stack([We1[l][2 * H + NG:].astype(f32)
                           for l in range(L)])
    etw_all = _tc_etw(etab, w1w_stack)

    tab, tabg = _tc_init(remap_pad, ntf_pad, pos_pad, w0, w1, w2, bn)

    for l in range(L):
        w1s = We1[l][:H].astype(f32)
        w1d = We1[l][H:2 * H].astype(f32)
        w1r = jnp.pad(We1[l][2 * H:2 * H + NG].astype(f32),
                      ((0, NGP - NG), (0, 0)))
        etw = etw_all[l]
        b1 = be1[l].astype(f32)[None, :]
        w2l = We2[l].astype(f32)
        b2 = be2[l].astype(f32)[None, :]
        wx1 = Wx1[l].astype(f32)
        bx1l = bx1[l].astype(f32)[None, :]
        wx2row = Wx2[l].astype(f32).reshape(1, H)
        bx2s = bx2[l].astype(f32).reshape(1, 1)
        wh1h = Wh1[l][:H].astype(f32)
        wh1a = Wh1[l][H:].astype(f32)
        bh1l = bh1[l].astype(f32)[None, :]
        wh2 = Wh2[l].astype(f32)
        bh2l = bh2[l].astype(f32)[None, :]

        parts = []
        for k in range(CH):
            gath = _sc_gather(tabg, gidx_ch[k])
            msg = _tc_edges(gath, et_ch[k], mu_row, etw,
                            w1s, w1d, w1r, b1, w2l, b2,
                            wx1, bx1l, wx2row, bx2s)
            parts.append(_sc_scatter(msg, dst_ch[k], zeros_pack))
        tab, tabg = _tc_nodes(tab, parts[0], parts[1],
                              wh1h, wh1a, bh1l, wh2, bh2l)

    wo2row = Wo2.astype(f32).reshape(1, H)
    bo2s = bo2.astype(f32).reshape(1, 1)
    out_row = _tc_readout(tab, bat_f, Wo1.astype(f32),
                          bo1.astype(f32)[None, :], wo2row, bo2s)
    out = out_row.reshape(B, 1)
    x_out = tab[:N, H:H + 3]
    return (out, x_out)
